# Initial kernel scaffold; baseline (speedup 1.0000x reference)
import jax
import jax.numpy as jnp
from jax import lax
from jax.experimental import pallas as pl
from jax.experimental.pallas import tpu as pltpu
from jax.experimental.pallas import tpu_sc as plsc

N = 10000
R = 46
B = 30
H = 16
C = 8
E = 640000

NC = 2
NS = 16
NW = NC * NS

GSZ = 128
GPW = 160
EPW = GPW * GSZ
EP = NW * EPW
SUP = 2048
GPS = SUP // GSZ
NSUP = EPW // SUP

NR = N * R
DEGR = 28800
DPT = DEGR // NS
APT = N // NS

_MESH = plsc.VectorSubcoreMesh(core_axis_name="c", subcore_axis_name="s")


def _zero_rows(ref, nrows):
    z = jnp.zeros((16,), jnp.float32)

    def body(i, _):
        ref[i, :] = z
        return 0

    lax.fori_loop(0, nrows, body, 0)


def _deg_body(dst_hbm, et_hbm, deg_out, dstb, etb, krow, kmodb, oneh, zbuf,
              deg_s, sem):
    cid = lax.axis_index("c")
    sid = lax.axis_index("s")
    wid = cid * NS + sid
    ii = lax.iota(jnp.int32, 16)
    c15 = jnp.full((16,), 15, jnp.int32)
    c4 = jnp.full((16,), 4, jnp.int32)

    _zero_rows(zbuf, DPT)
    pltpu.sync_copy(zbuf, deg_s.at[pl.ds(sid * DPT, DPT)])
    _zero_rows(oneh, GSZ)
    plsc.subcore_barrier()

    def super_body(sc, _):
        base = wid * EPW + sc * SUP
        pltpu.sync_copy(dst_hbm.at[pl.ds(base, SUP)], dstb)
        pltpu.sync_copy(et_hbm.at[pl.ds(base, SUP)], etb)

        def group_body(g, _):
            gb = g * GSZ
            for k in range(8):
                off = gb + k * 16
                d = dstb[pl.ds(off, 16)]
                e = etb[pl.ds(off, 16)]
                key = d * R + e
                krow[0, pl.ds(k * 16, 16)] = lax.shift_right_logical(key, c4)
                km = jnp.bitwise_and(key, c15)
                kmodb[pl.ds(k * 16, 16)] = km
                pos = base + gb + k * 16 + ii
                val = jnp.where(pos < E, 1.0, 0.0).astype(jnp.float32)
                plsc.store_scatter(oneh, [k * 16 + ii, km], val)
            pltpu.sync_copy(oneh, deg_s.at[krow.at[0]], add=True)
            zf = jnp.zeros((16,), jnp.float32)
            for k in range(8):
                km = kmodb[pl.ds(k * 16, 16)]
                plsc.store_scatter(oneh, [k * 16 + ii, km], zf)
            return 0

        lax.fori_loop(0, GPS, group_body, 0)
        return 0

    lax.fori_loop(0, NSUP, super_body, 0)
    plsc.subcore_barrier()
    pltpu.sync_copy(deg_s.at[pl.ds(sid * DPT, DPT)],
                    deg_out.at[cid, pl.ds(sid * DPT, DPT)])


_deg_kernel = pl.kernel(
    _deg_body,
    out_type=jax.ShapeDtypeStruct((NC, DEGR, 16), jnp.float32),
    mesh=_MESH,
    scratch_types=[
        pltpu.VMEM((SUP,), jnp.int32),
        pltpu.VMEM((SUP,), jnp.int32),
        pltpu.VMEM((1, GSZ), jnp.int32),
        pltpu.VMEM((GSZ,), jnp.int32),
        pltpu.VMEM((GSZ, 16), jnp.float32),
        pltpu.VMEM((DPT, 16), jnp.float32),
        pltpu.VMEM_SHARED((DEGR, 16), jnp.float32),
        pltpu.SemaphoreType.DMA,
    ],
)


def _edge_body(src_hbm, et_hbm, dst_hbm, tab_hbm, nrm_in, agg_out, nrm_out,
               srcb, etb, dstb, idx1, krow, kmodb, dstw, nrmsb, rows, recb,
               zbuf, acc_s, sem, *, layer1):
    cid = lax.axis_index("c")
    sid = lax.axis_index("s")
    wid = cid * NS + sid
    ii = lax.iota(jnp.int32, 16)
    c15 = jnp.full((16,), 15, jnp.int32)
    c4 = jnp.full((16,), 4, jnp.int32)

    _zero_rows(zbuf, APT)
    pltpu.sync_copy(zbuf, acc_s.at[pl.ds(sid * APT, APT)])
    plsc.subcore_barrier()

    def super_body(sc, _):
        base = wid * EPW + sc * SUP
        pltpu.sync_copy(src_hbm.at[pl.ds(base, SUP)], srcb)
        pltpu.sync_copy(et_hbm.at[pl.ds(base, SUP)], etb)
        pltpu.sync_copy(dst_hbm.at[pl.ds(base, SUP)], dstb)
        if not layer1:
            pltpu.sync_copy(nrm_in.at[pl.ds(base, SUP)], nrmsb)

        def group_body(g, _):
            gb = g * GSZ
            for k in range(8):
                off = gb + k * 16
                s_ = srcb[pl.ds(off, 16)]
                e_ = etb[pl.ds(off, 16)]
                d_ = dstb[pl.ds(off, 16)]
                idx1[0, pl.ds(k * 16, 16)] = e_ * N + s_
                dstw[0, pl.ds(k * 16, 16)] = d_
                if layer1:
                    key = d_ * R + e_
                    krow[0, pl.ds(k * 16, 16)] = lax.shift_right_logical(key,
                                                                         c4)
                    kmodb[pl.ds(k * 16, 16)] = jnp.bitwise_and(key, c15)
            pltpu.async_copy(tab_hbm.at[idx1.at[0]], rows, sem).wait()
            if layer1:
                pltpu.async_copy(nrm_in.at[krow.at[0]], recb, sem).wait()
            for k in range(8):
                rowv = k * 16 + ii
                if layer1:
                    km = kmodb[pl.ds(k * 16, 16)]
                    rv = plsc.load_gather(recb, [rowv, km])
                    pos = base + gb + k * 16 + ii
                    nv = jnp.where(pos < E, rv, 0.0)
                    nrmsb[pl.ds(gb + k * 16, 16)] = nv
                else:
                    nv = nrmsb[pl.ds(gb + k * 16, 16)]
                for f in range(16):
                    ff = jnp.full((16,), f, jnp.int32)
                    w = plsc.load_gather(rows, [rowv, ff])
                    plsc.store_scatter(rows, [rowv, ff], w * nv)
            pltpu.sync_copy(rows, acc_s.at[dstw.at[0]], add=True)
            return 0

        lax.fori_loop(0, GPS, group_body, 0)
        if layer1:
            pltpu.sync_copy(nrmsb, nrm_out.at[pl.ds(base, SUP)])
        return 0

    lax.fori_loop(0, NSUP, super_body, 0)
    plsc.subcore_barrier()
    pltpu.sync_copy(acc_s.at[pl.ds(sid * APT, APT)],
                    agg_out.at[cid, pl.ds(sid * APT, APT)])


def _make_edge_kernel(layer1):
    if layer1:
        def body(src, et, dst, tab, rec, agg, nrm, *scr):
            _edge_body(src, et, dst, tab, rec, agg, nrm, *scr, layer1=True)
        out_type = (jax.ShapeDtypeStruct((NC, N, 16), jnp.float32),
                    jax.ShapeDtypeStruct((EP,), jnp.float32))
    else:
        def body(src, et, dst, tab, nrm, agg, *scr):
            _edge_body(src, et, dst, tab, nrm, agg, None, *scr, layer1=False)
        out_type = jax.ShapeDtypeStruct((NC, N, 16), jnp.float32)
    return pl.kernel(
        body,
        out_type=out_type,
        mesh=_MESH,
        scratch_types=[
            pltpu.VMEM((SUP,), jnp.int32),
            pltpu.VMEM((SUP,), jnp.int32),
            pltpu.VMEM((SUP,), jnp.int32),
            pltpu.VMEM((1, GSZ), jnp.int32),
            pltpu.VMEM((1, GSZ), jnp.int32),
            pltpu.VMEM((GSZ,), jnp.int32),
            pltpu.VMEM((1, GSZ), jnp.int32),
            pltpu.VMEM((SUP,), jnp.float32),
            pltpu.VMEM((GSZ, 16), jnp.float32),
            pltpu.VMEM((GSZ, 16), jnp.float32),
            pltpu.VMEM((APT, 16), jnp.float32),
            pltpu.VMEM_SHARED((N, 16), jnp.float32),
            pltpu.SemaphoreType.DMA,
        ],
    )


_l1_kernel = _make_edge_kernel(True)
_l2_kernel = _make_edge_kernel(False)


_CK = 6400


def _w1_body(c_ref, b_ref, o_ref):
    o_ref[...] = jnp.dot(c_ref[...], b_ref[...],
                         preferred_element_type=jnp.float32)


def _recip_body(d_ref, o_ref):
    d = d_ref[0] + d_ref[1]
    o_ref[...] = 1.0 / jnp.maximum(d, 1.0)


def _h_body(a_ref, r_ref, b_ref, o_ref):
    o_ref[...] = jnp.maximum(a_ref[0] + a_ref[1] + r_ref[...] + b_ref[...],
                             0.0)


def _w2_body(c_ref, b_ref, o_ref):
    o_ref[...] = jnp.dot(c_ref[...], b_ref[...],
                         preferred_element_type=jnp.float32)


def _y_body(h_ref, w_ref, o_ref):
    o_ref[...] = jnp.dot(h_ref[...], w_ref[0],
                         preferred_element_type=jnp.float32)[None]


def _fin_body(a_ref, h_ref, rt_ref, b_ref, o_ref):
    a = a_ref[0, :, :C] + a_ref[1, :, :C]
    x = a + jnp.dot(h_ref[...], rt_ref[...],
                    preferred_element_type=jnp.float32) + b_ref[...]
    m = jnp.max(x, axis=1, keepdims=True)
    ex = jnp.exp(x - m)
    lse = jnp.log(jnp.sum(ex, axis=1, keepdims=True)) + m
    o_ref[...] = x - lse


def kernel(edge_index, edge_type, bases1, comp1, root1, bias1, bases2, comp2,
           root2, bias2):
    src = edge_index[0].astype(jnp.int32)
    dst = edge_index[1].astype(jnp.int32)
    et = edge_type.astype(jnp.int32)
    pad = EP - E
    srcp = jnp.pad(src, (0, pad))
    dstp = jnp.pad(dst, (0, pad))
    etp = jnp.pad(et, (0, pad))

    deg = _deg_kernel(dstp, etp)

    w1f = pl.pallas_call(
        _w1_body,
        grid=(N * H // _CK,),
        in_specs=[pl.BlockSpec((R, B), lambda i: (0, 0)),
                  pl.BlockSpec((B, _CK), lambda i: (0, i))],
        out_specs=pl.BlockSpec((R, _CK), lambda i: (0, i)),
        out_shape=jax.ShapeDtypeStruct((R, N * H), jnp.float32),
    )(comp1, bases1.reshape(B, N * H))
    w1t = w1f.reshape(R * N, H)

    recip = pl.pallas_call(
        _recip_body,
        out_shape=jax.ShapeDtypeStruct((DEGR, 16), jnp.float32),
    )(deg)

    agg1, normp = _l1_kernel(srcp, etp, dstp, w1t, recip)

    h = pl.pallas_call(
        _h_body,
        out_shape=jax.ShapeDtypeStruct((N, H), jnp.float32),
    )(agg1, root1, bias1.reshape(1, H))

    w2f = pl.pallas_call(
        _w2_body,
        out_shape=jax.ShapeDtypeStruct((R, H * C), jnp.float32),
    )(comp2, bases2.reshape(B, H * C))
    w2p = jnp.pad(w2f.reshape(R, H, C), ((0, 0), (0, 0), (0, 16 - C)))

    y = pl.pallas_call(
        _y_body,
        grid=(R,),
        in_specs=[pl.BlockSpec((N, H), lambda i: (0, 0)),
                  pl.BlockSpec((1, H, 16), lambda i: (i, 0, 0))],
        out_specs=pl.BlockSpec((1, N, 16), lambda i: (i, 0, 0)),
        out_shape=jax.ShapeDtypeStruct((R, N, 16), jnp.float32),
    )(h, w2p)
    yt = y.reshape(R * N, 16)

    agg2 = _l2_kernel(srcp, etp, dstp, yt, normp)

    out = pl.pallas_call(
        _fin_body,
        out_shape=jax.ShapeDtypeStruct((N, C), jnp.float32),
    )(agg2, h, root2, bias2.reshape(1, C))
    return out


# speedup vs baseline: 9.7646x; 9.7646x over previous
import jax
import jax.numpy as jnp
from jax import lax
from jax.experimental import pallas as pl
from jax.experimental.pallas import tpu as pltpu
from jax.experimental.pallas import tpu_sc as plsc

N = 10000
R = 46
B = 30
H = 16
C = 8
E = 640000

NC = 2
NS = 16
NW = NC * NS

GSZ = 128
GPW = 160
EPW = GPW * GSZ
EP = NW * EPW
SUP = 2048
GPS = SUP // GSZ
NSUP = EPW // SUP

NR = N * R
DEGR = 28800
DPT = DEGR // NS
APT = N // NS

_MESH = plsc.VectorSubcoreMesh(core_axis_name="c", subcore_axis_name="s",
                               num_cores=NC, num_subcores=NS)


def _zero_rows(ref, nrows):
    z = jnp.zeros((16,), jnp.float32)

    def body(i, _):
        ref[i, :] = z
        return 0

    lax.fori_loop(0, nrows, body, 0)


def _deg_body(dst_hbm, et_hbm, deg_out, dstb, etb, krow, kmodb, oneh, zbuf,
              deg_s, sem):
    cid = lax.axis_index("c")
    sid = lax.axis_index("s")
    wid = cid * NS + sid
    ii = lax.iota(jnp.int32, 16)
    c15 = jnp.full((16,), 15, jnp.int32)
    c4 = jnp.full((16,), 4, jnp.int32)

    _zero_rows(zbuf, DPT)
    pltpu.sync_copy(zbuf, deg_s.at[pl.ds(sid * DPT, DPT)])
    _zero_rows(oneh, GSZ)
    plsc.subcore_barrier()

    def super_body(sc, _):
        base = wid * EPW + sc * SUP
        pltpu.sync_copy(dst_hbm.at[pl.ds(base, SUP)], dstb)
        pltpu.sync_copy(et_hbm.at[pl.ds(base, SUP)], etb)

        def group_body(g, _):
            gb = g * GSZ
            for k in range(8):
                off = gb + k * 16
                d = dstb[pl.ds(off, 16)]
                e = etb[pl.ds(off, 16)]
                key = d * R + e
                krow[0, pl.ds(k * 16, 16)] = lax.shift_right_logical(key, c4)
                km = jnp.bitwise_and(key, c15)
                kmodb[pl.ds(k * 16, 16)] = km
                pos = base + gb + k * 16 + ii
                val = jnp.where(pos < E, 1.0, 0.0).astype(jnp.float32)
                plsc.store_scatter(oneh, [k * 16 + ii, km], val)
            pltpu.sync_copy(oneh, deg_s.at[krow.at[0]], add=True)
            zf = jnp.zeros((16,), jnp.float32)
            for k in range(8):
                km = kmodb[pl.ds(k * 16, 16)]
                plsc.store_scatter(oneh, [k * 16 + ii, km], zf)
            return 0

        lax.fori_loop(0, GPS, group_body, 0)
        return 0

    lax.fori_loop(0, NSUP, super_body, 0)
    plsc.subcore_barrier()
    pltpu.sync_copy(deg_s.at[pl.ds(sid * DPT, DPT)],
                    deg_out.at[cid, pl.ds(sid * DPT, DPT)])


_deg_kernel = pl.kernel(
    _deg_body,
    out_type=jax.ShapeDtypeStruct((NC, DEGR, 16), jnp.float32),
    mesh=_MESH,
    compiler_params=pltpu.CompilerParams(needs_layout_passes=False, use_tc_tiling_on_sc=False),
    scratch_types=[
        pltpu.VMEM((SUP,), jnp.int32),
        pltpu.VMEM((SUP,), jnp.int32),
        pltpu.VMEM((1, GSZ), jnp.int32),
        pltpu.VMEM((GSZ,), jnp.int32),
        pltpu.VMEM((GSZ, 16), jnp.float32),
        pltpu.VMEM((DPT, 16), jnp.float32),
        pltpu.VMEM_SHARED((DEGR, 16), jnp.float32),
        pltpu.SemaphoreType.DMA,
    ],
)


def _edge_body(src_hbm, et_hbm, dst_hbm, tab_hbm, nrm_in, agg_out, nrm_out,
               srcb, etb, dstb, idx1, krow, kmodb, dstw, nrmsb, rows, recb,
               zbuf, acc_s, sem, *, layer1):
    cid = lax.axis_index("c")
    sid = lax.axis_index("s")
    wid = cid * NS + sid
    ii = lax.iota(jnp.int32, 16)
    c15 = jnp.full((16,), 15, jnp.int32)
    c4 = jnp.full((16,), 4, jnp.int32)

    _zero_rows(zbuf, APT)
    pltpu.sync_copy(zbuf, acc_s.at[pl.ds(sid * APT, APT)])
    plsc.subcore_barrier()

    def super_body(sc, _):
        base = wid * EPW + sc * SUP
        pltpu.sync_copy(src_hbm.at[pl.ds(base, SUP)], srcb)
        pltpu.sync_copy(et_hbm.at[pl.ds(base, SUP)], etb)
        pltpu.sync_copy(dst_hbm.at[pl.ds(base, SUP)], dstb)
        if not layer1:
            pltpu.sync_copy(nrm_in.at[pl.ds(base, SUP)], nrmsb)

        def group_body(g, _):
            gb = g * GSZ
            for k in range(8):
                off = gb + k * 16
                s_ = srcb[pl.ds(off, 16)]
                e_ = etb[pl.ds(off, 16)]
                d_ = dstb[pl.ds(off, 16)]
                idx1[0, pl.ds(k * 16, 16)] = e_ * N + s_
                dstw[0, pl.ds(k * 16, 16)] = d_
                if layer1:
                    key = d_ * R + e_
                    krow[0, pl.ds(k * 16, 16)] = lax.shift_right_logical(key,
                                                                         c4)
                    kmodb[pl.ds(k * 16, 16)] = jnp.bitwise_and(key, c15)
            pltpu.async_copy(tab_hbm.at[idx1.at[0]], rows, sem).wait()
            if layer1:
                pltpu.async_copy(nrm_in.at[krow.at[0]], recb, sem).wait()
            for k in range(8):
                rowv = k * 16 + ii
                if layer1:
                    km = kmodb[pl.ds(k * 16, 16)]
                    rv = plsc.load_gather(recb, [rowv, km])
                    pos = base + gb + k * 16 + ii
                    nv = jnp.where(pos < E, rv, 0.0)
                    nrmsb[pl.ds(gb + k * 16, 16)] = nv
                else:
                    nv = nrmsb[pl.ds(gb + k * 16, 16)]
                for f in range(16):
                    ff = jnp.full((16,), f, jnp.int32)
                    w = plsc.load_gather(rows, [rowv, ff])
                    plsc.store_scatter(rows, [rowv, ff], w * nv)
            pltpu.sync_copy(rows, acc_s.at[dstw.at[0]], add=True)
            return 0

        lax.fori_loop(0, GPS, group_body, 0)
        if layer1:
            pltpu.sync_copy(nrmsb, nrm_out.at[pl.ds(base, SUP)])
        return 0

    lax.fori_loop(0, NSUP, super_body, 0)
    plsc.subcore_barrier()
    pltpu.sync_copy(acc_s.at[pl.ds(sid * APT, APT)],
                    agg_out.at[cid, pl.ds(sid * APT, APT)])


def _make_edge_kernel(layer1):
    if layer1:
        def body(src, et, dst, tab, rec, agg, nrm, *scr):
            _edge_body(src, et, dst, tab, rec, agg, nrm, *scr, layer1=True)
        out_type = (jax.ShapeDtypeStruct((NC, N, 16), jnp.float32),
                    jax.ShapeDtypeStruct((EP,), jnp.float32))
    else:
        def body(src, et, dst, tab, nrm, agg, *scr):
            _edge_body(src, et, dst, tab, nrm, agg, None, *scr, layer1=False)
        out_type = jax.ShapeDtypeStruct((NC, N, 16), jnp.float32)
    return pl.kernel(
        body,
        out_type=out_type,
        mesh=_MESH,
        compiler_params=pltpu.CompilerParams(needs_layout_passes=False, use_tc_tiling_on_sc=False),
        scratch_types=[
            pltpu.VMEM((SUP,), jnp.int32),
            pltpu.VMEM((SUP,), jnp.int32),
            pltpu.VMEM((SUP,), jnp.int32),
            pltpu.VMEM((1, GSZ), jnp.int32),
            pltpu.VMEM((1, GSZ), jnp.int32),
            pltpu.VMEM((GSZ,), jnp.int32),
            pltpu.VMEM((1, GSZ), jnp.int32),
            pltpu.VMEM((SUP,), jnp.float32),
            pltpu.VMEM((GSZ, 16), jnp.float32),
            pltpu.VMEM((GSZ, 16), jnp.float32),
            pltpu.VMEM((APT, 16), jnp.float32),
            pltpu.VMEM_SHARED((N, 16), jnp.float32),
            pltpu.SemaphoreType.DMA,
        ],
    )


_l1_kernel = _make_edge_kernel(True)
_l2_kernel = _make_edge_kernel(False)


_CK = 6400


def _w1_body(c_ref, b_ref, o_ref):
    o_ref[...] = jnp.dot(c_ref[...], b_ref[...],
                         preferred_element_type=jnp.float32)


def _recip_body(d_ref, o_ref):
    d = d_ref[0] + d_ref[1]
    o_ref[...] = 1.0 / jnp.maximum(d, 1.0)


def _h_body(a_ref, r_ref, b_ref, o_ref):
    o_ref[...] = jnp.maximum(a_ref[0] + a_ref[1] + r_ref[...] + b_ref[...],
                             0.0)


def _w2_body(c_ref, b_ref, o_ref):
    o_ref[...] = jnp.dot(c_ref[...], b_ref[...],
                         preferred_element_type=jnp.float32)


def _y_body(h_ref, w_ref, o_ref):
    o_ref[...] = jnp.dot(h_ref[...], w_ref[0],
                         preferred_element_type=jnp.float32)[None]


def _fin_body(a_ref, h_ref, rt_ref, b_ref, o_ref):
    a = a_ref[0, :, :C] + a_ref[1, :, :C]
    x = a + jnp.dot(h_ref[...], rt_ref[...],
                    preferred_element_type=jnp.float32) + b_ref[...]
    m = jnp.max(x, axis=1, keepdims=True)
    ex = jnp.exp(x - m)
    lse = jnp.log(jnp.sum(ex, axis=1, keepdims=True)) + m
    o_ref[...] = x - lse


def kernel(edge_index, edge_type, bases1, comp1, root1, bias1, bases2, comp2,
           root2, bias2):
    src = edge_index[0].astype(jnp.int32)
    dst = edge_index[1].astype(jnp.int32)
    et = edge_type.astype(jnp.int32)
    pad = EP - E
    srcp = jnp.pad(src, (0, pad))
    dstp = jnp.pad(dst, (0, pad))
    etp = jnp.pad(et, (0, pad))

    deg = _deg_kernel(dstp, etp)

    w1f = pl.pallas_call(
        _w1_body,
        grid=(N * H // _CK,),
        in_specs=[pl.BlockSpec((R, B), lambda i: (0, 0)),
                  pl.BlockSpec((B, _CK), lambda i: (0, i))],
        out_specs=pl.BlockSpec((R, _CK), lambda i: (0, i)),
        out_shape=jax.ShapeDtypeStruct((R, N * H), jnp.float32),
    )(comp1, bases1.reshape(B, N * H))
    w1t = w1f.reshape(R * N, H)

    recip = pl.pallas_call(
        _recip_body,
        out_shape=jax.ShapeDtypeStruct((DEGR, 16), jnp.float32),
    )(deg)

    agg1, normp = _l1_kernel(srcp, etp, dstp, w1t, recip)

    h = pl.pallas_call(
        _h_body,
        out_shape=jax.ShapeDtypeStruct((N, H), jnp.float32),
    )(agg1, root1, bias1.reshape(1, H))

    w2f = pl.pallas_call(
        _w2_body,
        out_shape=jax.ShapeDtypeStruct((R, H * C), jnp.float32),
    )(comp2, bases2.reshape(B, H * C))
    w2p = jnp.pad(w2f.reshape(R, H, C), ((0, 0), (0, 0), (0, 16 - C)))

    y = pl.pallas_call(
        _y_body,
        grid=(R,),
        in_specs=[pl.BlockSpec((N, H), lambda i: (0, 0)),
                  pl.BlockSpec((1, H, 16), lambda i: (i, 0, 0))],
        out_specs=pl.BlockSpec((1, N, 16), lambda i: (i, 0, 0)),
        out_shape=jax.ShapeDtypeStruct((R, N, 16), jnp.float32),
    )(h, w2p)
    yt = y.reshape(R * N, 16)

    agg2 = _l2_kernel(srcp, etp, dstp, yt, normp)

    out = pl.pallas_call(
        _fin_body,
        out_shape=jax.ShapeDtypeStruct((N, C), jnp.float32),
    )(agg2, h, root2, bias2.reshape(1, C))
    return out


# speedup vs baseline: 14.3588x; 1.4705x over previous
import jax
import jax.numpy as jnp
from jax import lax
from jax.experimental import pallas as pl
from jax.experimental.pallas import tpu as pltpu
from jax.experimental.pallas import tpu_sc as plsc

N = 10000
R = 46
B = 30
H = 16
C = 8
E = 640000

NC = 2
NS = 16
NW = NC * NS

GSZ = 128
GPW = 160
EPW = GPW * GSZ
EP = NW * EPW

NR = N * R
DEGR = 28800
DPT = DEGR // NS
APT = N // NS

MEGA = 10240
GPM = MEGA // GSZ
NMEGA = EPW // MEGA
NBUF = 8
RING = GPM // NBUF

_MESH = plsc.VectorSubcoreMesh(core_axis_name="c", subcore_axis_name="s",
                               num_cores=NC, num_subcores=NS)
_CPARAMS = pltpu.CompilerParams(needs_layout_passes=False,
                                use_tc_tiling_on_sc=False)


def _zero_rows(ref, nrows):
    z = jnp.zeros((16,), jnp.float32)

    def body(i, _):
        ref[i, :] = z
        return 0

    lax.fori_loop(0, nrows, body, 0)


def _deg_body(dst_hbm, et_hbm, deg_out, dstb, etb, krow128, kmodb, oneh,
              deg_s):
    cid = lax.axis_index("c")
    sid = lax.axis_index("s")
    wid = cid * NS + sid
    ii = lax.iota(jnp.int32, 16)
    c15 = jnp.full((16,), 15, jnp.int32)
    c4 = jnp.full((16,), 4, jnp.int32)

    _zero_rows(oneh, GSZ)
    for q in range(14):
        pltpu.sync_copy(oneh, deg_s.at[pl.ds(sid * DPT + q * GSZ, GSZ)])
    pltpu.sync_copy(oneh.at[pl.ds(0, DPT - 14 * GSZ)],
                    deg_s.at[pl.ds(sid * DPT + 14 * GSZ, DPT - 14 * GSZ)])
    plsc.subcore_barrier()

    for m in range(NMEGA):
        base = wid * EPW + m * MEGA
        base_eff = jnp.minimum(base, E - MEGA)
        pltpu.sync_copy(dst_hbm.at[pl.ds(base_eff, MEGA)], dstb)
        pltpu.sync_copy(et_hbm.at[pl.ds(base_eff, MEGA)], etb)

        def group_body(g, _):
            gb = g * GSZ
            for k in range(8):
                off = gb + k * 16
                d = dstb[pl.ds(off, 16)]
                e = etb[pl.ds(off, 16)]
                key = d * R + e
                krow128[pl.ds(k * 16, 16)] = lax.shift_right_logical(key, c4)
                km = jnp.bitwise_and(key, c15)
                kmodb[pl.ds(k * 16, 16)] = km
                p = base_eff + off + ii
                ok = jnp.logical_and(p >= base, p < E)
                val = jnp.where(ok, 1.0, 0.0).astype(jnp.float32)
                plsc.store_scatter(oneh, [k * 16 + ii, km], val)
            pltpu.sync_copy(oneh, deg_s.at[krow128], add=True)
            zf = jnp.zeros((16,), jnp.float32)
            for k in range(8):
                km = kmodb[pl.ds(k * 16, 16)]
                plsc.store_scatter(oneh, [k * 16 + ii, km], zf)
            return 0

        lax.fori_loop(0, GPM, group_body, 0)
    plsc.subcore_barrier()
    pltpu.sync_copy(deg_s.at[pl.ds(sid * DPT, DPT)],
                    deg_out.at[cid, pl.ds(sid * DPT, DPT)])


_deg_kernel = pl.kernel(
    _deg_body,
    out_type=jax.ShapeDtypeStruct((NC, DEGR, 16), jnp.float32),
    mesh=_MESH,
    compiler_params=_CPARAMS,
    scratch_types=[
        pltpu.VMEM((MEGA,), jnp.int32),
        pltpu.VMEM((MEGA,), jnp.int32),
        pltpu.VMEM((GSZ,), jnp.int32),
        pltpu.VMEM((GSZ,), jnp.int32),
        pltpu.VMEM((GSZ, 16), jnp.float32),
        pltpu.VMEM_SHARED((DEGR, 16), jnp.float32),
    ],
)


def _edge_body(src_hbm, et_hbm, dst_hbm, tab_hbm, nrm_in, agg_out, nrm_out,
               srcb, etb, dstb, idx1, krow, nrmsb, rows, recb, dstw, gsem,
               acc_s, *, layer1):
    cid = lax.axis_index("c")
    sid = lax.axis_index("s")
    wid = cid * NS + sid
    ii = lax.iota(jnp.int32, 16)
    c15 = jnp.full((16,), 15, jnp.int32)
    c4 = jnp.full((16,), 4, jnp.int32)

    _zero_rows(rows[0], GSZ)
    for q in range(4):
        pltpu.sync_copy(rows[0], acc_s.at[pl.ds(sid * APT + q * GSZ, GSZ)])
    pltpu.sync_copy(rows[0].at[pl.ds(0, APT - 4 * GSZ)],
                    acc_s.at[pl.ds(sid * APT + 4 * GSZ, APT - 4 * GSZ)])
    plsc.subcore_barrier()

    def fire(g, j):
        pltpu.async_copy(tab_hbm.at[idx1.at[pl.ds(g * GSZ, GSZ)]],
                         rows[j], gsem[j])
        if layer1:
            pltpu.async_copy(nrm_in.at[krow.at[pl.ds(g * GSZ, GSZ)]],
                             recb[j], gsem[j])

    def drain(j):
        pltpu.make_async_copy(tab_hbm.at[pl.ds(0, GSZ)], rows[j],
                              gsem[j]).wait()
        if layer1:
            pltpu.make_async_copy(nrm_in.at[pl.ds(0, GSZ)], recb[j],
                                  gsem[j]).wait()

    def mega_body(m, _):
        base = wid * EPW + m * MEGA
        base_eff = jnp.minimum(base, E - MEGA)
        slot = wid * EPW + m * MEGA
        pltpu.sync_copy(src_hbm.at[pl.ds(base_eff, MEGA)], srcb)
        pltpu.sync_copy(et_hbm.at[pl.ds(base_eff, MEGA)], etb)
        pltpu.sync_copy(dst_hbm.at[pl.ds(base_eff, MEGA)], dstb)
        if not layer1:
            pltpu.sync_copy(nrm_in.at[pl.ds(slot, MEGA)], nrmsb)

        def pro_body(g, _):
            gb = g * GSZ
            for k in range(8):
                off = gb + k * 16
                s_ = srcb[pl.ds(off, 16)]
                e_ = etb[pl.ds(off, 16)]
                idx1[pl.ds(off, 16)] = e_ * N + s_
                if layer1:
                    d_ = dstb[pl.ds(off, 16)]
                    key = d_ * R + e_
                    krow[pl.ds(off, 16)] = lax.shift_right_logical(key, c4)
            return 0

        lax.fori_loop(0, GPM, pro_body, 0)

        for j in range(NBUF):
            fire(j, j)

        def ring_body(t, _):
            for j in range(NBUF):
                g = t * NBUF + j
                gb = g * GSZ
                drain(j)
                for k in range(8):
                    off = gb + k * 16
                    rowv = k * 16 + ii
                    d_ = dstb[pl.ds(off, 16)]
                    dstw[j][pl.ds(k * 16, 16)] = d_
                    if layer1:
                        e_ = etb[pl.ds(off, 16)]
                        key = d_ * R + e_
                        km = jnp.bitwise_and(key, c15)
                        rv = plsc.load_gather(recb[j], [rowv, km])
                        p = base_eff + off + ii
                        ok = jnp.logical_and(p >= base, p < E)
                        nv = jnp.where(ok, rv, 0.0)
                        nrmsb[pl.ds(off, 16)] = nv
                    else:
                        nv = nrmsb[pl.ds(off, 16)]
                    for f in range(16):
                        ff = jnp.full((16,), f, jnp.int32)
                        w = plsc.load_gather(rows[j], [rowv, ff])
                        plsc.store_scatter(rows[j], [rowv, ff], w * nv)
                pltpu.sync_copy(rows[j], acc_s.at[dstw[j]], add=True)

                @pl.when(t < RING - 1)
                def _():
                    fire((t + 1) * NBUF + j, j)
            return 0

        lax.fori_loop(0, RING, ring_body, 0)
        if layer1:
            pltpu.sync_copy(nrmsb, nrm_out.at[pl.ds(slot, MEGA)])
        return 0

    lax.fori_loop(0, NMEGA, mega_body, 0)
    plsc.subcore_barrier()
    pltpu.sync_copy(acc_s.at[pl.ds(sid * APT, APT)],
                    agg_out.at[cid, pl.ds(sid * APT, APT)])


def _make_edge_kernel(layer1):
    if layer1:
        def body(src, et, dst, tab, rec, agg, nrm, *scr):
            srcb, etb, dstb, idx1, krow, nrmsb = scr[:6]
            rows = scr[6:6 + NBUF]
            recb = scr[6 + NBUF:6 + 2 * NBUF]
            dstw = scr[6 + 2 * NBUF:6 + 3 * NBUF]
            gsem = scr[6 + 3 * NBUF:6 + 4 * NBUF]
            acc_s = scr[6 + 4 * NBUF]
            _edge_body(src, et, dst, tab, rec, agg, nrm,
                       srcb, etb, dstb, idx1, krow, nrmsb, rows, recb, dstw,
                       gsem, acc_s, layer1=True)
        out_type = (jax.ShapeDtypeStruct((NC, N, 16), jnp.float32),
                    jax.ShapeDtypeStruct((EP,), jnp.float32))
    else:
        def body(src, et, dst, tab, nrm, agg, *scr):
            srcb, etb, dstb, idx1, krow, nrmsb = scr[:6]
            rows = scr[6:6 + NBUF]
            recb = [None] * NBUF
            dstw = scr[6 + NBUF:6 + 2 * NBUF]
            gsem = scr[6 + 2 * NBUF:6 + 3 * NBUF]
            acc_s = scr[6 + 3 * NBUF]
            _edge_body(src, et, dst, tab, nrm, agg, None,
                       srcb, etb, dstb, idx1, krow, nrmsb, rows, recb, dstw,
                       gsem, acc_s, layer1=False)
        out_type = jax.ShapeDtypeStruct((NC, N, 16), jnp.float32)

    scratch = [
        pltpu.VMEM((MEGA,), jnp.int32),
        pltpu.VMEM((MEGA,), jnp.int32),
        pltpu.VMEM((MEGA,), jnp.int32),
        pltpu.VMEM((MEGA,), jnp.int32),
        pltpu.VMEM((MEGA,), jnp.int32),
        pltpu.VMEM((MEGA,), jnp.float32),
    ]
    scratch += [pltpu.VMEM((GSZ, 16), jnp.float32)] * NBUF
    if layer1:
        scratch += [pltpu.VMEM((GSZ, 16), jnp.float32)] * NBUF
    scratch += [pltpu.VMEM((GSZ,), jnp.int32)] * NBUF
    scratch += [pltpu.SemaphoreType.DMA] * NBUF
    scratch += [pltpu.VMEM_SHARED((N, 16), jnp.float32)]
    return pl.kernel(
        body,
        out_type=out_type,
        mesh=_MESH,
        compiler_params=_CPARAMS,
        scratch_types=scratch,
    )


_l1_kernel = _make_edge_kernel(True)
_l2_kernel = _make_edge_kernel(False)


_CK = 6400


def _w1_body(c_ref, b_ref, o_ref):
    o_ref[...] = jnp.dot(c_ref[...], b_ref[...],
                         preferred_element_type=jnp.float32)


def _recip_body(d_ref, o_ref):
    d = d_ref[0] + d_ref[1]
    o_ref[...] = 1.0 / jnp.maximum(d, 1.0)


def _h_body(a_ref, r_ref, b_ref, o_ref):
    o_ref[...] = jnp.maximum(a_ref[0] + a_ref[1] + r_ref[...] + b_ref[...],
                             0.0)


def _w2_body(c_ref, b_ref, o_ref):
    o_ref[...] = jnp.dot(c_ref[...], b_ref[...],
                         preferred_element_type=jnp.float32)


def _y_body(h_ref, w_ref, o_ref):
    o_ref[...] = jnp.dot(h_ref[...], w_ref[0],
                         preferred_element_type=jnp.float32)[None]


def _fin_body(a_ref, h_ref, rt_ref, b_ref, o_ref):
    a = a_ref[0, :, :C] + a_ref[1, :, :C]
    x = a + jnp.dot(h_ref[...], rt_ref[...],
                    preferred_element_type=jnp.float32) + b_ref[...]
    m = jnp.max(x, axis=1, keepdims=True)
    ex = jnp.exp(x - m)
    lse = jnp.log(jnp.sum(ex, axis=1, keepdims=True)) + m
    o_ref[...] = x - lse


def kernel(edge_index, edge_type, bases1, comp1, root1, bias1, bases2, comp2,
           root2, bias2):
    src = edge_index[0].astype(jnp.int32)
    dst = edge_index[1].astype(jnp.int32)
    et = edge_type.astype(jnp.int32)

    deg = _deg_kernel(dst, et)

    w1f = pl.pallas_call(
        _w1_body,
        grid=(N * H // _CK,),
        in_specs=[pl.BlockSpec((R, B), lambda i: (0, 0)),
                  pl.BlockSpec((B, _CK), lambda i: (0, i))],
        out_specs=pl.BlockSpec((R, _CK), lambda i: (0, i)),
        out_shape=jax.ShapeDtypeStruct((R, N * H), jnp.float32),
    )(comp1, bases1.reshape(B, N * H))
    w1t = w1f.reshape(R * N, H)

    recip = pl.pallas_call(
        _recip_body,
        out_shape=jax.ShapeDtypeStruct((DEGR, 16), jnp.float32),
    )(deg)

    agg1, normp = _l1_kernel(src, et, dst, w1t, recip)

    h = pl.pallas_call(
        _h_body,
        out_shape=jax.ShapeDtypeStruct((N, H), jnp.float32),
    )(agg1, root1, bias1.reshape(1, H))

    w2f = pl.pallas_call(
        _w2_body,
        out_shape=jax.ShapeDtypeStruct((R, H * C), jnp.float32),
    )(comp2, bases2.reshape(B, H * C))
    w2p = jnp.pad(w2f.reshape(R, H, C), ((0, 0), (0, 0), (0, 16 - C)))

    y = pl.pallas_call(
        _y_body,
        grid=(R,),
        in_specs=[pl.BlockSpec((N, H), lambda i: (0, 0)),
                  pl.BlockSpec((1, H, 16), lambda i: (i, 0, 0))],
        out_specs=pl.BlockSpec((1, N, 16), lambda i: (i, 0, 0)),
        out_shape=jax.ShapeDtypeStruct((R, N, 16), jnp.float32),
    )(h, w2p)
    yt = y.reshape(R * N, 16)

    agg2 = _l2_kernel(src, et, dst, yt, normp)

    out = pl.pallas_call(
        _fin_body,
        out_shape=jax.ShapeDtypeStruct((N, C), jnp.float32),
    )(agg2, h, root2, bias2.reshape(1, C))
    return out
